# baseline (speedup 1.0000x reference)
import functools

import jax
import jax.numpy as jnp
import numpy as np
from jax import lax
from jax.experimental import pallas as pl
from jax.experimental.pallas import tpu as pltpu
from jax.experimental.pallas import tpu_sc as plsc

_N0 = 2500.0
_LANES = 16
_TC_LANES = 128


def _sc_half(x2, patt, in_rows, *, num_cores, num_subcores):
    num_workers = num_cores * num_subcores
    lanes = x2.shape[1]
    m = patt.shape[0]
    irows_per_w = in_rows // num_workers
    orows_per_w = in_rows // 2 // num_workers
    vecs_per_w = orows_per_w * lanes // _LANES
    vecs_per_mrow = m // _LANES

    mesh = plsc.VectorSubcoreMesh(
        core_axis_name="c", subcore_axis_name="s",
        num_cores=num_cores, num_subcores=num_subcores,
    )

    @functools.partial(
        pl.kernel,
        out_type=jax.ShapeDtypeStruct((in_rows // 2, lanes), jnp.float32),
        mesh=mesh,
        scratch_types=[
            pltpu.VMEM((irows_per_w, lanes), jnp.float32),
            pltpu.VMEM((m,), jnp.float32),
            pltpu.VMEM((orows_per_w, lanes), jnp.float32),
        ],
        compiler_params=pltpu.CompilerParams(needs_layout_passes=False),
    )
    def run(x_hbm, patt_hbm, out_hbm, x_v, patt_v, out_v):
        wid = lax.axis_index("s") * num_cores + lax.axis_index("c")
        pltpu.sync_copy(x_hbm.at[pl.ds(wid * irows_per_w, irows_per_w)], x_v)
        pltpu.sync_copy(patt_hbm, patt_v)

        even_iota = 2 * lax.iota(jnp.int32, _LANES)
        odd_iota = even_iota + 1
        zero = jnp.zeros((_LANES,), jnp.int32)
        scale = jnp.float32(2.0 / _N0)

        @plsc.parallel_loop(0, vecs_per_w, 1, unroll=8)
        def body(v):
            irow = zero + lax.shift_right_logical(v, 2)
            cbase = lax.shift_left(lax.bitwise_and(v, 3), 5)
            even = plsc.load_gather(x_v, [irow, cbase + even_iota])
            odd = plsc.load_gather(x_v, [irow, cbase + odd_iota])
            pm = lax.bitwise_and(v, vecs_per_mrow - 1) * _LANES
            p = patt_v[pl.ds(pm, _LANES)]
            orow = lax.shift_right_logical(v, 3)
            ocol = lax.shift_left(lax.bitwise_and(v, 7), 4)
            out_v[orow, pl.ds(ocol, _LANES)] = (even - odd) * scale - p

        pltpu.sync_copy(out_v, out_hbm.at[pl.ds(wid * orows_per_w, orows_per_w)])

    return run(x2, patt)


def _tc_body(x_ref, w_ref, p_ref, o_ref):
    xb = x_ref[:, 0, :]
    blk_b, two_m = xb.shape
    kdim = w_ref.shape[0]
    z = jnp.reshape(xb, (blk_b * two_m // kdim, kdim))
    y = lax.dot_general(
        z, w_ref[...], (((1,), (0,)), ((), ())),
        precision=lax.Precision.HIGHEST,
        preferred_element_type=jnp.float32,
    )
    reps = y.shape[0] // p_ref.shape[0]
    p = jnp.reshape(
        jnp.broadcast_to(p_ref[...][None], (reps,) + p_ref.shape),
        y.shape)
    o_ref[...] = y * jnp.float32(2.0 / _N0) - p


def _tc_half(x, patt, batch_lo):
    bs, cs, two_m = x.shape
    m = patt.shape[0]
    kdim = 2 * _TC_LANES
    nbatch = bs - batch_lo
    rows_out = nbatch * cs * two_m // kdim
    wnp = np.zeros((kdim, _TC_LANES), np.float32)
    wnp[2 * np.arange(_TC_LANES), np.arange(_TC_LANES)] = 1.0
    wnp[2 * np.arange(_TC_LANES) + 1, np.arange(_TC_LANES)] = -1.0
    wmat = jnp.asarray(wnp)
    patt2 = jnp.reshape(patt, (m // _TC_LANES, _TC_LANES))
    blk_b = 16
    rows_per_blk = blk_b * cs * two_m // kdim
    blk_lo = batch_lo // blk_b
    return pl.pallas_call(
        _tc_body,
        grid=(nbatch * cs // blk_b,),
        in_specs=[
            pl.BlockSpec((blk_b, 1, two_m), lambda i: (i + blk_lo, 0, 0)),
            pl.BlockSpec((kdim, _TC_LANES), lambda i: (0, 0)),
            pl.BlockSpec((m // _TC_LANES, _TC_LANES), lambda i: (0, 0)),
        ],
        out_specs=pl.BlockSpec((rows_per_blk, _TC_LANES), lambda i: (i, 0)),
        out_shape=jax.ShapeDtypeStruct((rows_out, _TC_LANES), jnp.float32),
    )(x, wmat, patt2)


def kernel(x, Patt, b, c, h, w):
    bs, cs, two_m = x.shape
    m = Patt.shape[0]
    patt = Patt.astype(jnp.float32)
    half = bs // 2
    info = plsc.get_sparse_core_info()
    x2 = jnp.reshape(x, (bs * cs * two_m // _TC_LANES, _TC_LANES))
    in_rows_half = half * cs * two_m // _TC_LANES
    out_rows_total = bs * cs * m // _TC_LANES
    sc_out = _sc_half(x2, patt, in_rows_half,
                      num_cores=info.num_cores,
                      num_subcores=info.num_subcores)
    tc_out = _tc_half(x, patt, half)
    out = jnp.concatenate([sc_out, tc_out], axis=0)
    return jnp.reshape(out, (bs, cs, m))


# speedup vs baseline: 1.0726x; 1.0726x over previous
import functools

import jax
import jax.numpy as jnp
from jax import lax
from jax.experimental import pallas as pl
from jax.experimental.pallas import tpu as pltpu
from jax.experimental.pallas import tpu_sc as plsc

_N0 = 2500.0
_LANES = 16
_VIEW_LANES = 128


def _preprocess_sc(x2, patt, *, num_cores, num_subcores):
    num_workers = num_cores * num_subcores
    in_rows, lanes = x2.shape
    m = patt.shape[0]
    out_rows = in_rows // 2
    irows_per_w = in_rows // num_workers
    orows_per_w = out_rows // num_workers
    vecs_per_w = orows_per_w * lanes // _LANES
    vecs_per_mrow = m // _LANES
    assert m & (m - 1) == 0 and lanes == _VIEW_LANES

    mesh = plsc.VectorSubcoreMesh(
        core_axis_name="c", subcore_axis_name="s",
        num_cores=num_cores, num_subcores=num_subcores,
    )

    @functools.partial(
        pl.kernel,
        out_type=jax.ShapeDtypeStruct((out_rows, lanes), jnp.float32),
        mesh=mesh,
        scratch_types=[
            pltpu.VMEM((irows_per_w, lanes), jnp.float32),
            pltpu.VMEM((m,), jnp.float32),
            pltpu.VMEM((orows_per_w, lanes), jnp.float32),
        ],
        compiler_params=pltpu.CompilerParams(needs_layout_passes=False),
    )
    def run(x_hbm, patt_hbm, out_hbm, x_v, patt_v, out_v):
        wid = lax.axis_index("s") * num_cores + lax.axis_index("c")
        pltpu.sync_copy(x_hbm.at[pl.ds(wid * irows_per_w, irows_per_w)], x_v)
        pltpu.sync_copy(patt_hbm, patt_v)

        even_iota = 2 * lax.iota(jnp.int32, _LANES)
        odd_iota = even_iota + 1
        zero = jnp.zeros((_LANES,), jnp.int32)
        scale = jnp.float32(2.0 / _N0)

        @plsc.parallel_loop(0, vecs_per_w, 1, unroll=8)
        def body(v):
            irow = zero + lax.shift_right_logical(v, 2)
            cbase = lax.shift_left(lax.bitwise_and(v, 3), 5)
            even = plsc.load_gather(x_v, [irow, cbase + even_iota])
            odd = plsc.load_gather(x_v, [irow, cbase + odd_iota])
            pm = lax.bitwise_and(v, vecs_per_mrow - 1) * _LANES
            p = patt_v[pl.ds(pm, _LANES)]
            orow = lax.shift_right_logical(v, 3)
            ocol = lax.shift_left(lax.bitwise_and(v, 7), 4)
            out_v[orow, pl.ds(ocol, _LANES)] = (even - odd) * scale - p

        pltpu.sync_copy(out_v, out_hbm.at[pl.ds(wid * orows_per_w, orows_per_w)])

    return run(x2, patt)


def kernel(x, Patt, b, c, h, w):
    bs, cs, two_m = x.shape
    m = Patt.shape[0]
    x2 = jnp.reshape(x, (bs * cs * two_m // _VIEW_LANES, _VIEW_LANES))
    info = plsc.get_sparse_core_info()
    out = _preprocess_sc(x2, Patt.astype(jnp.float32),
                         num_cores=info.num_cores,
                         num_subcores=info.num_subcores)
    return jnp.reshape(out, (bs, cs, m))
